# Initial kernel scaffold; baseline (speedup 1.0000x reference)
import jax
import jax.numpy as jnp
from jax.experimental import pallas as pl


def kernel(x, w1, b1, w2, b2):
    raise NotImplementedError("write your pallas kernel here")


# speedup vs baseline: 1.0054x; 1.0054x over previous
import functools

import jax
import jax.numpy as jnp
from jax.experimental import pallas as pl
from jax.experimental.pallas import tpu as pltpu

_VMEM_LIMIT_BYTES = 48 * 1024 * 1024
_LANES = 128


def _cca_kernel(x_ref, w1_ref, b1_ref, w2_ref, b2_ref, o_ref, *, hw, nseg):
    x = x_ref[0].astype(jnp.float32)

    s = x[:, 0:_LANES]
    q = s * s
    for k in range(1, nseg):
        xk = x[:, k * _LANES:(k + 1) * _LANES]
        s = s + xk
        q = q + xk * xk

    inv_hw = 1.0 / hw
    mean = jnp.sum(s, axis=1, keepdims=True) * inv_hw
    ex2 = jnp.sum(q, axis=1, keepdims=True) * inv_hw
    var = jnp.maximum(ex2 - mean * mean, 0.0)
    y = jnp.sqrt(var) + mean

    z1 = jnp.sum(w1_ref[...] * y, axis=0, keepdims=True) + b1_ref[...]
    z1 = jnp.maximum(z1, 0.0)
    z2 = jnp.sum(w2_ref[...] * z1, axis=1, keepdims=True) + b2_ref[...]
    scale = 1.0 / (1.0 + jnp.exp(-z2))

    o_ref[0] = (x_ref[0] * scale.astype(o_ref.dtype)).astype(o_ref.dtype)


def kernel(x, w1, b1, w2, b2):
    N, C, H, W = x.shape
    Cmid = w1.shape[0]
    HW = H * W
    dtype = x.dtype

    xr = x.reshape(N, C, HW)
    w1t = jnp.transpose(w1[:, :, 0, 0], (1, 0))
    b1r = b1.reshape(1, Cmid)
    w2r = w2[:, :, 0, 0]
    b2r = b2.reshape(C, 1)

    assert HW % _LANES == 0, "spatial extent must be lane-aligned"
    nseg = HW // _LANES

    out = pl.pallas_call(
        functools.partial(_cca_kernel, hw=float(HW), nseg=nseg),
        out_shape=jax.ShapeDtypeStruct((N, C, HW), dtype),
        grid=(N,),
        in_specs=[
            pl.BlockSpec((1, C, HW), lambda n: (n, 0, 0)),
            pl.BlockSpec((C, Cmid), lambda n: (0, 0)),
            pl.BlockSpec((1, Cmid), lambda n: (0, 0)),
            pl.BlockSpec((C, Cmid), lambda n: (0, 0)),
            pl.BlockSpec((C, 1), lambda n: (0, 0)),
        ],
        out_specs=pl.BlockSpec((1, C, HW), lambda n: (n, 0, 0)),
        compiler_params=pltpu.CompilerParams(
            dimension_semantics=("parallel",),
            vmem_limit_bytes=_VMEM_LIMIT_BYTES),
    )(xr, w1t, b1r, w2r, b2r)
    return out.reshape(N, C, H, W)


# speedup vs baseline: 1.1283x; 1.1223x over previous
import functools

import jax
import jax.numpy as jnp
from jax.experimental import pallas as pl
from jax.experimental.pallas import tpu as pltpu

_VMEM_LIMIT_BYTES = 48 * 1024 * 1024
_LANES = 128
_BATCH_BLK = 8


def _cca_kernel(x_ref, w1_ref, b1_ref, w2_ref, b2_ref, o_ref, *, hw, nseg, bb):
    inv_hw = 1.0 / hw
    for b in range(bb):
        x = x_ref[b]

        s = x[:, 0:_LANES]
        q = s * s
        for k in range(1, nseg):
            xk = x[:, k * _LANES:(k + 1) * _LANES]
            s = s + xk
            q = q + xk * xk

        mean = jnp.sum(s, axis=1, keepdims=True) * inv_hw
        ex2 = jnp.sum(q, axis=1, keepdims=True) * inv_hw
        var = jnp.maximum(ex2 - mean * mean, 0.0)
        y = jnp.sqrt(var) + mean

        z1 = jnp.sum(w1_ref[...] * y, axis=0, keepdims=True) + b1_ref[...]
        z1 = jnp.maximum(z1, 0.0)
        z2 = jnp.sum(w2_ref[...] * z1, axis=1, keepdims=True) + b2_ref[...]
        scale = 1.0 / (1.0 + jnp.exp(-z2))

        o_ref[b] = x * scale


def kernel(x, w1, b1, w2, b2):
    N, C, H, W = x.shape
    Cmid = w1.shape[0]
    HW = H * W
    dtype = x.dtype

    xr = x.reshape(N, C, HW)
    w1t = jnp.transpose(w1[:, :, 0, 0], (1, 0))
    b1r = b1.reshape(1, Cmid)
    w2r = w2[:, :, 0, 0]
    b2r = b2.reshape(C, 1)

    assert HW % _LANES == 0, "spatial extent must be lane-aligned"
    nseg = HW // _LANES
    bb = _BATCH_BLK
    while N % bb != 0:
        bb //= 2

    out = pl.pallas_call(
        functools.partial(_cca_kernel, hw=float(HW), nseg=nseg, bb=bb),
        out_shape=jax.ShapeDtypeStruct((N, C, HW), dtype),
        grid=(N // bb,),
        in_specs=[
            pl.BlockSpec((bb, C, HW), lambda n: (n, 0, 0)),
            pl.BlockSpec((C, Cmid), lambda n: (0, 0)),
            pl.BlockSpec((1, Cmid), lambda n: (0, 0)),
            pl.BlockSpec((C, Cmid), lambda n: (0, 0)),
            pl.BlockSpec((C, 1), lambda n: (0, 0)),
        ],
        out_specs=pl.BlockSpec((bb, C, HW), lambda n: (n, 0, 0)),
        compiler_params=pltpu.CompilerParams(
            dimension_semantics=("arbitrary",),
            vmem_limit_bytes=_VMEM_LIMIT_BYTES),
    )(xr, w1t, b1r, w2r, b2r)
    return out.reshape(N, C, H, W)


# speedup vs baseline: 3.1985x; 2.8348x over previous
import functools

import jax
import jax.numpy as jnp
from jax.experimental import pallas as pl
from jax.experimental.pallas import tpu as pltpu

_VMEM_LIMIT_BYTES = 48 * 1024 * 1024
_BATCH_BLK = 4


def _cca_kernel(x_ref, w1_ref, b1_ref, w2_ref, b2_ref, o_ref, *, hw, bb):
    inv_hw = 1.0 / hw
    for b in range(bb):
        x = x_ref[b]

        s = jnp.sum(x, axis=(1, 2), keepdims=True)
        q = jnp.sum(x * x, axis=(1, 2), keepdims=True)
        mean = s * inv_hw
        ex2 = q * inv_hw
        var = jnp.maximum(ex2 - mean * mean, 0.0)
        y = (jnp.sqrt(var) + mean)[:, :, 0]

        z1 = jnp.sum(w1_ref[...] * y, axis=0, keepdims=True) + b1_ref[...]
        z1 = jnp.maximum(z1, 0.0)
        z2 = jnp.sum(w2_ref[...] * z1, axis=1, keepdims=True) + b2_ref[...]
        scale = 1.0 / (1.0 + jnp.exp(-z2))

        o_ref[b] = x * scale[:, :, None]


def kernel(x, w1, b1, w2, b2):
    N, C, H, W = x.shape
    Cmid = w1.shape[0]
    dtype = x.dtype

    w1t = jnp.transpose(w1[:, :, 0, 0], (1, 0))
    b1r = b1.reshape(1, Cmid)
    w2r = w2[:, :, 0, 0]
    b2r = b2.reshape(C, 1)

    bb = _BATCH_BLK
    while N % bb != 0:
        bb //= 2

    out = pl.pallas_call(
        functools.partial(_cca_kernel, hw=float(H * W), bb=bb),
        out_shape=jax.ShapeDtypeStruct((N, C, H, W), dtype),
        grid=(N // bb,),
        in_specs=[
            pl.BlockSpec((bb, C, H, W), lambda n: (n, 0, 0, 0)),
            pl.BlockSpec((C, Cmid), lambda n: (0, 0)),
            pl.BlockSpec((1, Cmid), lambda n: (0, 0)),
            pl.BlockSpec((C, Cmid), lambda n: (0, 0)),
            pl.BlockSpec((C, 1), lambda n: (0, 0)),
        ],
        out_specs=pl.BlockSpec((bb, C, H, W), lambda n: (n, 0, 0, 0)),
        compiler_params=pltpu.CompilerParams(
            dimension_semantics=("arbitrary",),
            vmem_limit_bytes=_VMEM_LIMIT_BYTES),
    )(x, w1t, b1r, w2r, b2r)
    return out
